# Initial kernel scaffold; baseline (speedup 1.0000x reference)
import jax
import jax.numpy as jnp
from jax.experimental import pallas as pl


def kernel(feat, in_idx):
    raise NotImplementedError("write your pallas kernel here")


# speedup vs baseline: 2.1072x; 2.1072x over previous
import functools

import jax
import jax.numpy as jnp
from jax import lax
from jax.experimental import pallas as pl
from jax.experimental.pallas import tpu as pltpu
from jax.experimental.pallas import tpu_sc as plsc

_L = 16
_CH = 400
_SUB = 80
_NC = 2
_NS = 16


def kernel(feat, in_idx):
    n_in, d = feat.shape
    n_out = in_idx.shape[0]
    idx32 = in_idx.astype(jnp.int32)

    nw = _NC * _NS
    n_chunks = n_out // _CH
    iters = (n_chunks + nw - 1) // nw
    spread_mask = (1 << (n_in.bit_length() - 1)) - 1

    mesh = plsc.VectorSubcoreMesh(
        core_axis_name="c", subcore_axis_name="s",
        num_cores=_NC, num_subcores=_NS,
    )

    @functools.partial(
        pl.kernel,
        out_type=jax.ShapeDtypeStruct((n_out, d), jnp.float32),
        mesh=mesh,
        scratch_types=[
            pltpu.VMEM((_CH,), jnp.int32),
            pltpu.VMEM((_CH,), jnp.int32),
            pltpu.VMEM((_CH, d), jnp.float32),
            pltpu.SemaphoreType.DMA,
        ],
        compiler_params=pltpu.CompilerParams(
            needs_layout_passes=False,
            use_tc_tiling_on_sc=False,
        ),
    )
    def body(feat_hbm, idx_hbm, out_hbm, idx_v, sidx_v, rows_v, sem):
        wid = lax.axis_index("s") * _NC + lax.axis_index("c")
        lane = lax.iota(jnp.int32, _L)
        zeros_v = jnp.zeros((_L,), jnp.float32)

        def chunk_body(i, carry):
            chunk = i * nw + wid

            @pl.when(chunk < n_chunks)
            def _():
                base = chunk * _CH
                pltpu.sync_copy(idx_hbm.at[pl.ds(base, _CH)], idx_v)

                def remap(g, c):
                    v = idx_v[pl.ds(g * _L, _L)]
                    spread = (lane + (base + g * _L)) & spread_mask
                    sidx_v[pl.ds(g * _L, _L)] = jnp.where(v >= 0, v, spread)
                    return c

                lax.fori_loop(0, _CH // _L, remap, 0)

                copies = []
                for j in range(0, _CH, _SUB):
                    copies.append(pltpu.async_copy(
                        feat_hbm.at[sidx_v.at[pl.ds(j, _SUB)]],
                        rows_v.at[pl.ds(j, _SUB)],
                        sem,
                    ))
                for cp in copies:
                    cp.wait()

                def maskz(g, c):
                    v = idx_v[pl.ds(g * _L, _L)]
                    invalid = v < 0
                    rows = lane + g * _L
                    for col in range(d):
                        cols = jnp.full((_L,), col, jnp.int32)
                        plsc.store_scatter(rows_v, [rows, cols], zeros_v,
                                           mask=invalid)
                    return c

                lax.fori_loop(0, _CH // _L, maskz, 0)
                pltpu.sync_copy(rows_v, out_hbm.at[pl.ds(base, _CH)])

            return carry

        lax.fori_loop(0, iters, chunk_body, 0)

    return body(feat, idx32)


# speedup vs baseline: 2.4879x; 1.1807x over previous
import functools

import jax
import jax.numpy as jnp
from jax import lax
from jax.experimental import pallas as pl
from jax.experimental.pallas import tpu as pltpu
from jax.experimental.pallas import tpu_sc as plsc

_L = 16
_CH = 800
_SUB = 80
_NB = 3
_NC = 2
_NS = 16


def kernel(feat, in_idx):
    n_in, d = feat.shape
    n_out = in_idx.shape[0]
    idx32 = in_idx.astype(jnp.int32)

    nw = _NC * _NS
    n_chunks = n_out // _CH
    iters = (n_chunks + nw - 1) // nw
    spread_mask = (1 << (n_in.bit_length() - 1)) - 1

    mesh = plsc.VectorSubcoreMesh(
        core_axis_name="c", subcore_axis_name="s",
        num_cores=_NC, num_subcores=_NS,
    )

    @functools.partial(
        pl.kernel,
        out_type=jax.ShapeDtypeStruct((n_out, d), jnp.float32),
        mesh=mesh,
        scratch_types=[
            pltpu.VMEM((_NB, _CH), jnp.int32),
            pltpu.VMEM((_NB, _CH), jnp.int32),
            pltpu.VMEM((_NB, _CH, d), jnp.float32),
            pltpu.SemaphoreType.DMA((_NB,)),
            pltpu.SemaphoreType.DMA((_NB,)),
            pltpu.SemaphoreType.DMA((_NB,)),
        ],
        compiler_params=pltpu.CompilerParams(
            needs_layout_passes=False,
            use_tc_tiling_on_sc=False,
        ),
    )
    def body(feat_hbm, idx_hbm, out_hbm, idx_v, sidx_v, rows_v, isem, gsem,
             osem):
        wid = lax.axis_index("s") * _NC + lax.axis_index("c")
        lane = lax.iota(jnp.int32, _L)
        zeros_v = jnp.zeros((_L,), jnp.float32)

        def step(i, carry):
            chunk_a = i * nw + wid

            @pl.when(jnp.logical_and(i < iters, chunk_a < n_chunks))
            def _():
                s = lax.rem(i, _NB)
                pltpu.async_copy(
                    idx_hbm.at[pl.ds(chunk_a * _CH, _CH)],
                    idx_v.at[s], isem.at[s])

            ib = i - 1
            chunk_b = ib * nw + wid

            @pl.when(jnp.logical_and(
                jnp.logical_and(ib >= 0, ib < iters), chunk_b < n_chunks))
            def _():
                s = lax.rem(ib, _NB)
                pltpu.make_async_copy(
                    idx_hbm.at[pl.ds(0, _CH)], idx_v.at[s],
                    isem.at[s]).wait()
                base = chunk_b * _CH

                def remap(g, c):
                    v = idx_v[s, pl.ds(g * _L, _L)]
                    spread = (lane + (base + g * _L)) & spread_mask
                    sidx_v[s, pl.ds(g * _L, _L)] = jnp.where(v >= 0, v,
                                                             spread)
                    return c

                lax.fori_loop(0, _CH // _L, remap, 0)

                io = i - 4
                chunk_o = io * nw + wid

                @pl.when(jnp.logical_and(io >= 0, chunk_o < n_chunks))
                def _():
                    pltpu.make_async_copy(
                        rows_v.at[s], out_hbm.at[pl.ds(0, _CH)],
                        osem.at[s]).wait()

                for j in range(0, _CH, _SUB):
                    pltpu.async_copy(
                        feat_hbm.at[sidx_v.at[s, pl.ds(j, _SUB)]],
                        rows_v.at[s, pl.ds(j, _SUB)],
                        gsem.at[s])

            ic = i - 2
            chunk_c = ic * nw + wid

            @pl.when(jnp.logical_and(
                jnp.logical_and(ic >= 0, ic < iters), chunk_c < n_chunks))
            def _():
                s = lax.rem(ic, _NB)
                pltpu.make_async_copy(
                    feat_hbm.at[pl.ds(0, _CH)], rows_v.at[s],
                    gsem.at[s]).wait()

                def maskz(g, c):
                    v = idx_v[s, pl.ds(g * _L, _L)]
                    invalid = v < 0
                    rows = lane + g * _L
                    for col in range(d):
                        cols = jnp.full((_L,), col, jnp.int32)
                        plsc.store_scatter(rows_v.at[s], [rows, cols],
                                           zeros_v, mask=invalid)
                    return c

                lax.fori_loop(0, _CH // _L, maskz, 0)
                pltpu.async_copy(
                    rows_v.at[s],
                    out_hbm.at[pl.ds(chunk_c * _CH, _CH)],
                    osem.at[s])

            return carry

        lax.fori_loop(0, iters + 2, step, 0)

        def drain(i, carry):
            it = iters - 1 - i
            chunk_t = it * nw + wid

            @pl.when(jnp.logical_and(it >= 0, chunk_t < n_chunks))
            def _():
                s = lax.rem(it, _NB)
                pltpu.make_async_copy(
                    rows_v.at[s], out_hbm.at[pl.ds(0, _CH)],
                    osem.at[s]).wait()

            return carry

        lax.fori_loop(0, min(3, iters), drain, 0)

    return body(feat, idx32)
